# baseline (speedup 1.0000x reference)
import functools

import jax
import jax.numpy as jnp
from jax import lax
from jax.experimental import pallas as pl
from jax.experimental.pallas import tpu as pltpu

F32 = jnp.float32
BF16 = jnp.bfloat16
VMEM_LIMIT = 58 * 1024 * 1024


def _round_up(x, m):
    return ((x + m - 1) // m) * m


def _pick_tile(T, cands=(160, 100, 80, 64, 50, 40, 32, 25, 20, 16, 10, 8, 5, 4, 2, 1)):
    for tt in cands:
        if T % tt == 0:
            return tt
    return 1


def _enc_kernel(p1_ref, p2_ref, w1_ref, w2_ref, lng_ref, lnb_ref,
                ws_ref, w5_ref, b5_ref, o_ref):
    h = jnp.dot(p1_ref[...], w1_ref[...], preferred_element_type=F32)
    h = h + jnp.dot(p2_ref[...], w2_ref[...], preferred_element_type=F32)
    h = jnp.maximum(h, 0.0)

    def ln(v, i):
        mu = jnp.mean(v, axis=-1, keepdims=True)
        var = jnp.mean((v - mu) ** 2, axis=-1, keepdims=True)
        return (v - mu) * lax.rsqrt(var + 1e-5) * lng_ref[i] + lnb_ref[i]

    for i in range(4):
        h = jnp.maximum(ln(h, i), 0.0)
        h = jnp.dot(h.astype(BF16), ws_ref[i], preferred_element_type=F32)
    h = jnp.maximum(ln(h, 4), 0.0)
    o_ref[...] = jnp.dot(h.astype(BF16), w5_ref[...],
                         preferred_element_type=F32) + b5_ref[...]


def _uni4_kernel(x_ref, wih0_ref, wihr_ref, whh_ref, b_ref, o_ref,
                 h_sc, c_sc, act_sc, g_sc, *, tile_t, n_layers):
    H = h_sc.shape[-1]
    Bb = h_sc.shape[-2]

    @pl.when(pl.program_id(0) == 0)
    def _init():
        h_sc[...] = jnp.zeros_like(h_sc)
        c_sc[...] = jnp.zeros_like(c_sc)

    def run_layer(l, in_ref, wih_ref_l, store_out):
        Din = in_ref.shape[-1]
        g = jnp.dot(in_ref[...].reshape(tile_t * Bb, Din).astype(BF16),
                    wih_ref_l[...], preferred_element_type=F32)
        g_sc[...] = g.reshape(tile_t, Bb, 4 * H) + b_ref[l]

        unroll = 4 if tile_t % 4 == 0 else 1

        def step(j, carry):
            hb, c = carry
            for u in range(unroll):
                t = j * unroll + u
                gates = g_sc[t] + jnp.dot(hb, whh_ref[l],
                                          preferred_element_type=F32)
                i_g = jax.nn.sigmoid(gates[:, 0 * H:1 * H])
                f_g = jax.nn.sigmoid(gates[:, 1 * H:2 * H])
                g_g = jnp.tanh(gates[:, 2 * H:3 * H])
                o_g = jax.nn.sigmoid(gates[:, 3 * H:4 * H])
                c = f_g * c + i_g * g_g
                h = o_g * jnp.tanh(c)
                hb = h.astype(BF16)
                store_out(t, hb)
            return hb, c

        h_fin, c_fin = lax.fori_loop(
            0, tile_t // unroll, step,
            (h_sc[l].astype(BF16), c_sc[l]))
        h_sc[l] = h_fin.astype(F32)
        c_sc[l] = c_fin

    def to_act(t, hb):
        act_sc[t] = hb

    def to_out(t, hb):
        o_ref[t] = hb

    run_layer(0, x_ref, wih0_ref, to_act if n_layers > 1 else to_out)
    for l in range(1, n_layers):
        run_layer(l, act_sc, wihr_ref.at[l - 1],
                  to_out if l == n_layers - 1 else to_act)


def _bi_kernel(x_ref, wih_ref, whh_ref, b_ref, o_ref, h_sc, c_sc, g_sc,
               *, tile_t, B):
    H = h_sc.shape[-1]
    d = pl.program_id(0)

    @pl.when(pl.program_id(1) == 0)
    def _init():
        h_sc[...] = jnp.zeros_like(h_sc)
        c_sc[...] = jnp.zeros_like(c_sc)

    g = jnp.dot(x_ref[...], wih_ref[0], preferred_element_type=F32)
    g_sc[...] = g + b_ref[0]

    unroll = 4 if tile_t % 4 == 0 else 1

    def step(j, carry):
        hb, c = carry
        for u in range(unroll):
            i = j * unroll + u
            t = jnp.where(d == 0, i, tile_t - 1 - i)
            row = t * B
            gates = g_sc[pl.ds(row, B)] + jnp.dot(hb, whh_ref[0],
                                                  preferred_element_type=F32)
            i_g = jax.nn.sigmoid(gates[:, 0 * H:1 * H])
            f_g = jax.nn.sigmoid(gates[:, 1 * H:2 * H])
            g_g = jnp.tanh(gates[:, 2 * H:3 * H])
            o_g = jax.nn.sigmoid(gates[:, 3 * H:4 * H])
            c = f_g * c + i_g * g_g
            h = o_g * jnp.tanh(c)
            hb = h.astype(BF16)
            o_ref[pl.ds(row, B), :] = h.astype(o_ref.dtype)
        return hb, c

    h_fin, c_fin = lax.fori_loop(
        0, tile_t // unroll, step,
        (h_sc[...].astype(BF16), c_sc[...]))
    h_sc[...] = h_fin.astype(F32)
    c_sc[...] = c_fin


def _bi_layer(x2d, wih_f, whh_f, b_f, wih_b, whh_b, b_b, *, T, B, tile_t,
              out_dtype=F32):
    Din = x2d.shape[-1]
    H = whh_f.shape[0]
    n_chunks = T // tile_t
    wih = jnp.stack([wih_f, wih_b]).astype(BF16)
    whh = jnp.stack([whh_f, whh_b]).astype(BF16)
    b = jnp.stack([b_f, b_b]).astype(F32)

    def cidx(d, i):
        return i + d * (n_chunks - 1 - 2 * i)

    return pl.pallas_call(
        functools.partial(_bi_kernel, tile_t=tile_t, B=B),
        grid=(2, n_chunks),
        in_specs=[
            pl.BlockSpec((tile_t * B, Din), lambda d, i: (cidx(d, i), 0)),
            pl.BlockSpec((1, Din, 4 * H), lambda d, i: (d, 0, 0)),
            pl.BlockSpec((1, H, 4 * H), lambda d, i: (d, 0, 0)),
            pl.BlockSpec((1, 1, 4 * H), lambda d, i: (d, 0, 0)),
        ],
        out_specs=pl.BlockSpec((tile_t * B, H), lambda d, i: (cidx(d, i), d)),
        out_shape=jax.ShapeDtypeStruct((T * B, 2 * H), out_dtype),
        scratch_shapes=[pltpu.VMEM((B, H), F32), pltpu.VMEM((B, H), F32),
                        pltpu.VMEM((tile_t * B, 4 * H), F32)],
        compiler_params=pltpu.CompilerParams(
            dimension_semantics=("arbitrary", "arbitrary"),
            vmem_limit_bytes=VMEM_LIMIT),
    )(x2d, wih, whh, b)


def kernel(mels, conv_w, ln_g, ln_b, enc_ws, enc_w5, enc_b5,
           rnn1_wih, rnn1_whh, rnn1_b,
           rnn2_wih, rnn2_whh, rnn2_b,
           rnn3_wih, rnn3_whh, rnn3_b,
           rnn4_wih, rnn4_whh, rnn4_b,
           eng1_f_wih, eng1_f_whh, eng1_f_b,
           eng1_b_wih, eng1_b_whh, eng1_b_b,
           eng2_f_wih, eng2_f_whh, eng2_f_b,
           eng2_b_wih, eng2_b_whh, eng2_b_b):
    stride, padding = 2, 1
    B, Cin, T = mels.shape
    Cout, _, K = conv_w.shape
    T_out = (T + 2 * padding - K) // stride + 1
    z_dim = enc_w5.shape[1]

    x_t = jnp.pad(mels, ((0, 0), (0, 0), (padding, padding))).transpose(0, 2, 1)
    p1 = x_t[:, :stride * T_out, :].reshape(B * T_out, 2 * Cin).astype(BF16)
    p2 = x_t[:, 2:2 + stride * T_out, :].reshape(B * T_out, 2 * Cin).astype(BF16)
    w1 = conv_w[:, :, 0:2].transpose(2, 1, 0).reshape(2 * Cin, Cout).astype(BF16)
    w2 = conv_w[:, :, 2:4].transpose(2, 1, 0).reshape(2 * Cin, Cout).astype(BF16)

    M = B * T_out
    tile_m = min(512, _round_up(M, 8))
    Mp = _round_up(M, tile_m)
    if Mp != M:
        p1 = jnp.pad(p1, ((0, Mp - M), (0, 0)))
        p2 = jnp.pad(p2, ((0, Mp - M), (0, 0)))

    z_flat = pl.pallas_call(
        _enc_kernel,
        out_shape=jax.ShapeDtypeStruct((Mp, z_dim), F32),
        grid=(Mp // tile_m,),
        in_specs=[
            pl.BlockSpec((tile_m, 2 * Cin), lambda i: (i, 0)),
            pl.BlockSpec((tile_m, 2 * Cin), lambda i: (i, 0)),
            pl.BlockSpec((2 * Cin, Cout), lambda i: (0, 0)),
            pl.BlockSpec((2 * Cin, Cout), lambda i: (0, 0)),
            pl.BlockSpec((5, 1, Cout), lambda i: (0, 0, 0)),
            pl.BlockSpec((5, 1, Cout), lambda i: (0, 0, 0)),
            pl.BlockSpec((4, Cout, Cout), lambda i: (0, 0, 0)),
            pl.BlockSpec((Cout, z_dim), lambda i: (0, 0)),
            pl.BlockSpec((1, z_dim), lambda i: (0, 0)),
        ],
        out_specs=pl.BlockSpec((tile_m, z_dim), lambda i: (i, 0)),
        compiler_params=pltpu.CompilerParams(
            dimension_semantics=("arbitrary",),
            vmem_limit_bytes=VMEM_LIMIT),
    )(p1, p2, w1, w2, ln_g, ln_b, enc_ws.astype(BF16),
      enc_w5.astype(BF16), enc_b5)

    z = z_flat[:M].reshape(B, T_out, z_dim)
    z_tm = z.transpose(1, 0, 2)

    H = rnn1_whh.shape[0]
    tile_t = _pick_tile(T_out)
    n_chunks = T_out // tile_t
    Bb = B

    wih0 = rnn1_wih.astype(BF16)
    wihr = jnp.stack([rnn2_wih, rnn3_wih, rnn4_wih]).astype(BF16)
    whh = jnp.stack([rnn1_whh, rnn2_whh, rnn3_whh, rnn4_whh]).astype(BF16)
    bias = jnp.stack([rnn1_b, rnn2_b, rnn3_b, rnn4_b]).astype(F32)

    c_tm = pl.pallas_call(
        functools.partial(_uni4_kernel, tile_t=tile_t, n_layers=4),
        grid=(n_chunks,),
        in_specs=[
            pl.BlockSpec((tile_t, Bb, z_dim), lambda i: (i, 0, 0)),
            pl.BlockSpec((z_dim, 4 * H), lambda i: (0, 0)),
            pl.BlockSpec((3, H, 4 * H), lambda i: (0, 0, 0)),
            pl.BlockSpec((4, H, 4 * H), lambda i: (0, 0, 0)),
            pl.BlockSpec((4, 1, 4 * H), lambda i: (0, 0, 0)),
        ],
        out_specs=pl.BlockSpec((tile_t, Bb, H), lambda i: (i, 0, 0)),
        out_shape=jax.ShapeDtypeStruct((T_out, B, H), BF16),
        scratch_shapes=[pltpu.VMEM((4, Bb, H), F32),
                        pltpu.VMEM((4, Bb, H), F32),
                        pltpu.VMEM((tile_t, Bb, H), BF16),
                        pltpu.VMEM((tile_t, Bb, 4 * H), F32)],
        compiler_params=pltpu.CompilerParams(
            dimension_semantics=("arbitrary",),
            vmem_limit_bytes=VMEM_LIMIT),
    )(z_tm.astype(BF16), wih0, wihr, whh, bias)

    tile_bi = _pick_tile(T_out, (100, 80, 64, 50, 40, 32, 25, 20, 16,
                                 10, 8, 5, 4, 2, 1))
    s1 = _bi_layer(c_tm.reshape(T_out * B, H),
                   eng1_f_wih, eng1_f_whh, eng1_f_b,
                   eng1_b_wih, eng1_b_whh, eng1_b_b,
                   T=T_out, B=B, tile_t=tile_bi, out_dtype=BF16)
    H1 = eng1_f_whh.shape[0]
    s2 = _bi_layer(s1,
                   eng2_f_wih, eng2_f_whh, eng2_f_b,
                   eng2_b_wih, eng2_b_whh, eng2_b_b,
                   T=T_out, B=B, tile_t=tile_bi)
    H2 = eng2_f_whh.shape[0]
    s = s2.reshape(T_out, B, 2 * H2).transpose(1, 2, 0)
    return z, z, s


# speedup vs baseline: 1.0062x; 1.0062x over previous
import functools

import jax
import jax.numpy as jnp
from jax import lax
from jax.experimental import pallas as pl
from jax.experimental.pallas import tpu as pltpu

F32 = jnp.float32
BF16 = jnp.bfloat16
VMEM_LIMIT = 48 * 1024 * 1024


def _round_up(x, m):
    return ((x + m - 1) // m) * m


def _pick_tile(T, cands=(100, 80, 64, 50, 40, 32, 25, 20, 16, 10, 8, 5, 4, 2, 1)):
    for tt in cands:
        if T % tt == 0:
            return tt
    return 1


def _enc_kernel(p1_ref, p2_ref, w1_ref, w2_ref, lng_ref, lnb_ref,
                ws_ref, w5_ref, b5_ref, o_ref):
    h = jnp.dot(p1_ref[...], w1_ref[...], preferred_element_type=F32)
    h = h + jnp.dot(p2_ref[...], w2_ref[...], preferred_element_type=F32)
    h = jnp.maximum(h, 0.0)

    def ln(v, i):
        mu = jnp.mean(v, axis=-1, keepdims=True)
        var = jnp.mean((v - mu) ** 2, axis=-1, keepdims=True)
        return (v - mu) * lax.rsqrt(var + 1e-5) * lng_ref[i] + lnb_ref[i]

    for i in range(4):
        h = jnp.maximum(ln(h, i), 0.0)
        h = jnp.dot(h.astype(BF16), ws_ref[i], preferred_element_type=F32)
    h = jnp.maximum(ln(h, 4), 0.0)
    o_ref[...] = jnp.dot(h.astype(BF16), w5_ref[...],
                         preferred_element_type=F32) + b5_ref[...]


def _uni4_kernel(x_ref, wih0_ref, wihr_ref, whh_ref, b_ref, o_ref,
                 h_sc, c_sc, act_sc, g_sc, *, tile_t, n_layers):
    H = h_sc.shape[-1]
    Bb = h_sc.shape[-2]

    @pl.when(pl.program_id(0) == 0)
    def _init():
        h_sc[...] = jnp.zeros_like(h_sc)
        c_sc[...] = jnp.zeros_like(c_sc)

    def run_layer(l, in_ref, wih_ref_l, store_out):
        Din = in_ref.shape[-1]
        g = jnp.dot(in_ref[...].reshape(tile_t * Bb, Din).astype(BF16),
                    wih_ref_l[...], preferred_element_type=F32)
        g_sc[...] = g.reshape(tile_t, Bb, 4 * H) + b_ref[l]

        unroll = 4 if tile_t % 4 == 0 else 1

        def step(j, carry):
            hb, c = carry
            for u in range(unroll):
                t = j * unroll + u
                gates = g_sc[t] + jnp.dot(hb, whh_ref[l],
                                          preferred_element_type=F32)
                i_g = jax.nn.sigmoid(gates[:, 0 * H:1 * H])
                f_g = jax.nn.sigmoid(gates[:, 1 * H:2 * H])
                g_g = jnp.tanh(gates[:, 2 * H:3 * H])
                o_g = jax.nn.sigmoid(gates[:, 3 * H:4 * H])
                c = f_g * c + i_g * g_g
                h = o_g * jnp.tanh(c)
                hb = h.astype(BF16)
                store_out(t, hb)
            return hb, c

        h_fin, c_fin = lax.fori_loop(
            0, tile_t // unroll, step,
            (h_sc[l].astype(BF16), c_sc[l]))
        h_sc[l] = h_fin.astype(F32)
        c_sc[l] = c_fin

    def to_act(t, hb):
        act_sc[t] = hb

    def to_out(t, hb):
        o_ref[t] = hb

    run_layer(0, x_ref, wih0_ref, to_act if n_layers > 1 else to_out)
    for l in range(1, n_layers):
        run_layer(l, act_sc, wihr_ref.at[l - 1],
                  to_out if l == n_layers - 1 else to_act)


def _bi_kernel(x_ref, wih_ref, whh_ref, b_ref, o_ref, h_sc, c_sc, g_sc,
               *, tile_t, B):
    H = h_sc.shape[-1]
    d = pl.program_id(0)

    @pl.when(pl.program_id(1) == 0)
    def _init():
        h_sc[...] = jnp.zeros_like(h_sc)
        c_sc[...] = jnp.zeros_like(c_sc)

    g = jnp.dot(x_ref[...], wih_ref[0], preferred_element_type=F32)
    g_sc[...] = g + b_ref[0]

    unroll = 4 if tile_t % 4 == 0 else 1

    def step(j, carry):
        hb, c = carry
        for u in range(unroll):
            i = j * unroll + u
            t = jnp.where(d == 0, i, tile_t - 1 - i)
            row = t * B
            gates = g_sc[pl.ds(row, B)] + jnp.dot(hb, whh_ref[0],
                                                  preferred_element_type=F32)
            i_g = jax.nn.sigmoid(gates[:, 0 * H:1 * H])
            f_g = jax.nn.sigmoid(gates[:, 1 * H:2 * H])
            g_g = jnp.tanh(gates[:, 2 * H:3 * H])
            o_g = jax.nn.sigmoid(gates[:, 3 * H:4 * H])
            c = f_g * c + i_g * g_g
            h = o_g * jnp.tanh(c)
            hb = h.astype(BF16)
            o_ref[pl.ds(row, B), :] = h.astype(o_ref.dtype)
        return hb, c

    h_fin, c_fin = lax.fori_loop(
        0, tile_t // unroll, step,
        (h_sc[...].astype(BF16), c_sc[...]))
    h_sc[...] = h_fin.astype(F32)
    c_sc[...] = c_fin


def _bi_layer(x2d, wih_f, whh_f, b_f, wih_b, whh_b, b_b, *, T, B, tile_t,
              out_dtype=F32):
    Din = x2d.shape[-1]
    H = whh_f.shape[0]
    n_chunks = T // tile_t
    wih = jnp.stack([wih_f, wih_b]).astype(BF16)
    whh = jnp.stack([whh_f, whh_b]).astype(BF16)
    b = jnp.stack([b_f, b_b]).astype(F32)

    def cidx(d, i):
        return i + d * (n_chunks - 1 - 2 * i)

    return pl.pallas_call(
        functools.partial(_bi_kernel, tile_t=tile_t, B=B),
        grid=(2, n_chunks),
        in_specs=[
            pl.BlockSpec((tile_t * B, Din), lambda d, i: (cidx(d, i), 0)),
            pl.BlockSpec((1, Din, 4 * H), lambda d, i: (d, 0, 0)),
            pl.BlockSpec((1, H, 4 * H), lambda d, i: (d, 0, 0)),
            pl.BlockSpec((1, 1, 4 * H), lambda d, i: (d, 0, 0)),
        ],
        out_specs=pl.BlockSpec((tile_t * B, H), lambda d, i: (cidx(d, i), d)),
        out_shape=jax.ShapeDtypeStruct((T * B, 2 * H), out_dtype),
        scratch_shapes=[pltpu.VMEM((B, H), F32), pltpu.VMEM((B, H), F32),
                        pltpu.VMEM((tile_t * B, 4 * H), F32)],
        compiler_params=pltpu.CompilerParams(
            dimension_semantics=("arbitrary", "arbitrary"),
            vmem_limit_bytes=VMEM_LIMIT),
    )(x2d, wih, whh, b)


def kernel(mels, conv_w, ln_g, ln_b, enc_ws, enc_w5, enc_b5,
           rnn1_wih, rnn1_whh, rnn1_b,
           rnn2_wih, rnn2_whh, rnn2_b,
           rnn3_wih, rnn3_whh, rnn3_b,
           rnn4_wih, rnn4_whh, rnn4_b,
           eng1_f_wih, eng1_f_whh, eng1_f_b,
           eng1_b_wih, eng1_b_whh, eng1_b_b,
           eng2_f_wih, eng2_f_whh, eng2_f_b,
           eng2_b_wih, eng2_b_whh, eng2_b_b):
    stride, padding = 2, 1
    B, Cin, T = mels.shape
    Cout, _, K = conv_w.shape
    T_out = (T + 2 * padding - K) // stride + 1
    z_dim = enc_w5.shape[1]

    x_t = jnp.pad(mels, ((0, 0), (0, 0), (padding, padding))).transpose(0, 2, 1)
    p1 = x_t[:, :stride * T_out, :].reshape(B * T_out, 2 * Cin).astype(BF16)
    p2 = x_t[:, 2:2 + stride * T_out, :].reshape(B * T_out, 2 * Cin).astype(BF16)
    w1 = conv_w[:, :, 0:2].transpose(2, 1, 0).reshape(2 * Cin, Cout).astype(BF16)
    w2 = conv_w[:, :, 2:4].transpose(2, 1, 0).reshape(2 * Cin, Cout).astype(BF16)

    M = B * T_out
    tile_m = min(512, _round_up(M, 8))
    Mp = _round_up(M, tile_m)
    if Mp != M:
        p1 = jnp.pad(p1, ((0, Mp - M), (0, 0)))
        p2 = jnp.pad(p2, ((0, Mp - M), (0, 0)))

    z_flat = pl.pallas_call(
        _enc_kernel,
        out_shape=jax.ShapeDtypeStruct((Mp, z_dim), F32),
        grid=(Mp // tile_m,),
        in_specs=[
            pl.BlockSpec((tile_m, 2 * Cin), lambda i: (i, 0)),
            pl.BlockSpec((tile_m, 2 * Cin), lambda i: (i, 0)),
            pl.BlockSpec((2 * Cin, Cout), lambda i: (0, 0)),
            pl.BlockSpec((2 * Cin, Cout), lambda i: (0, 0)),
            pl.BlockSpec((5, 1, Cout), lambda i: (0, 0, 0)),
            pl.BlockSpec((5, 1, Cout), lambda i: (0, 0, 0)),
            pl.BlockSpec((4, Cout, Cout), lambda i: (0, 0, 0)),
            pl.BlockSpec((Cout, z_dim), lambda i: (0, 0)),
            pl.BlockSpec((1, z_dim), lambda i: (0, 0)),
        ],
        out_specs=pl.BlockSpec((tile_m, z_dim), lambda i: (i, 0)),
        compiler_params=pltpu.CompilerParams(
            dimension_semantics=("arbitrary",),
            vmem_limit_bytes=VMEM_LIMIT),
    )(p1, p2, w1, w2, ln_g, ln_b, enc_ws.astype(BF16),
      enc_w5.astype(BF16), enc_b5)

    z = z_flat[:M].reshape(B, T_out, z_dim)
    z_tm = z.transpose(1, 0, 2)

    H = rnn1_whh.shape[0]
    tile_t = _pick_tile(T_out)
    n_chunks = T_out // tile_t
    Bb = B

    wih0 = rnn1_wih.astype(BF16)
    wihr = jnp.stack([rnn2_wih, rnn3_wih, rnn4_wih]).astype(BF16)
    whh = jnp.stack([rnn1_whh, rnn2_whh, rnn3_whh, rnn4_whh]).astype(BF16)
    bias = jnp.stack([rnn1_b, rnn2_b, rnn3_b, rnn4_b]).astype(F32)

    c_tm = pl.pallas_call(
        functools.partial(_uni4_kernel, tile_t=tile_t, n_layers=4),
        grid=(n_chunks,),
        in_specs=[
            pl.BlockSpec((tile_t, Bb, z_dim), lambda i: (i, 0, 0)),
            pl.BlockSpec((z_dim, 4 * H), lambda i: (0, 0)),
            pl.BlockSpec((3, H, 4 * H), lambda i: (0, 0, 0)),
            pl.BlockSpec((4, H, 4 * H), lambda i: (0, 0, 0)),
            pl.BlockSpec((4, 1, 4 * H), lambda i: (0, 0, 0)),
        ],
        out_specs=pl.BlockSpec((tile_t, Bb, H), lambda i: (i, 0, 0)),
        out_shape=jax.ShapeDtypeStruct((T_out, B, H), BF16),
        scratch_shapes=[pltpu.VMEM((4, Bb, H), F32),
                        pltpu.VMEM((4, Bb, H), F32),
                        pltpu.VMEM((tile_t, Bb, H), BF16),
                        pltpu.VMEM((tile_t, Bb, 4 * H), F32)],
        compiler_params=pltpu.CompilerParams(
            dimension_semantics=("arbitrary",),
            vmem_limit_bytes=VMEM_LIMIT),
    )(z_tm.astype(BF16), wih0, wihr, whh, bias)

    tile_bi = _pick_tile(T_out, (100, 80, 64, 50, 40, 32, 25, 20, 16,
                                 10, 8, 5, 4, 2, 1))
    s1 = _bi_layer(c_tm.reshape(T_out * B, H),
                   eng1_f_wih, eng1_f_whh, eng1_f_b,
                   eng1_b_wih, eng1_b_whh, eng1_b_b,
                   T=T_out, B=B, tile_t=tile_bi, out_dtype=BF16)
    H1 = eng1_f_whh.shape[0]
    s2 = _bi_layer(s1,
                   eng2_f_wih, eng2_f_whh, eng2_f_b,
                   eng2_b_wih, eng2_b_whh, eng2_b_b,
                   T=T_out, B=B, tile_t=tile_bi)
    H2 = eng2_f_whh.shape[0]
    s = s2.reshape(T_out, B, 2 * H2).transpose(1, 2, 0)
    return z, z, s
